# Initial kernel scaffold; baseline (speedup 1.0000x reference)
import jax
import jax.numpy as jnp
from jax.experimental import pallas as pl


def kernel(hidden_states, weight, e_score_correction_bias):
    raise NotImplementedError("write your pallas kernel here")


# speedup vs baseline: 1.0147x; 1.0147x over previous
import functools

import jax
import jax.numpy as jnp
from jax.experimental import pallas as pl

BSZ, SEQ, H = 2, 4096, 4096
E = 64
TOP_K = 8
N_GROUP = 8
GSIZE = E // N_GROUP
TOPK_GROUP = 4
ROUTED_SCALING = 2.5
MOE_LOSS_ALPHA = 0.001

BLK = 256
T = BSZ * SEQ
NBLK = T // BLK

_NEG_INF = float("-inf")


def _gate_kernel(x_ref, w_ref, bias_ref, idx_ref, tw_ref, cnt_ref, ssum_ref):
    x = x_ref[...]
    w = w_ref[...]
    logits = jax.lax.dot_general(
        x, w, dimension_numbers=(((1,), (1,)), ((), ())),
        preferred_element_type=jnp.float32)
    scores = jax.nn.sigmoid(logits)
    sc = scores + bias_ref[...]

    iota_g = jax.lax.broadcasted_iota(jnp.int32, (BLK, GSIZE), 1)
    iota_ng = jax.lax.broadcasted_iota(jnp.int32, (BLK, N_GROUP), 1)
    iota_e = jax.lax.broadcasted_iota(jnp.int32, (BLK, E), 1)

    gparts = []
    for g in range(N_GROUP):
        sub = sc[:, g * GSIZE:(g + 1) * GSIZE]
        m1 = jnp.max(sub, axis=1, keepdims=True)
        i1 = jnp.min(jnp.where(sub == m1, iota_g, GSIZE), axis=1, keepdims=True)
        sub2 = jnp.where(iota_g == i1, _NEG_INF, sub)
        m2 = jnp.max(sub2, axis=1, keepdims=True)
        gparts.append(m1 + m2)
    group_scores = jnp.concatenate(gparts, axis=1)

    keep = []
    for g in range(N_GROUP):
        gi = group_scores[:, g:g + 1]
        better = (group_scores > gi) | ((group_scores == gi) & (iota_ng < g))
        rank = jnp.sum(better.astype(jnp.int32), axis=1, keepdims=True)
        keep.append((rank < TOPK_GROUP).astype(jnp.float32))
    emask = jnp.concatenate(
        [jnp.broadcast_to(keep[g], (BLK, GSIZE)) for g in range(N_GROUP)],
        axis=1)

    tmp = jnp.where(emask > 0.5, sc, 0.0)

    idx_cols, w_cols = [], []
    selmask = jnp.zeros((BLK, E), jnp.float32)
    for _ in range(TOP_K):
        m = jnp.max(tmp, axis=1, keepdims=True)
        sel = jnp.min(jnp.where(tmp == m, iota_e, E), axis=1, keepdims=True)
        onehot = iota_e == sel
        idx_cols.append(sel)
        w_cols.append(jnp.sum(jnp.where(onehot, scores, 0.0), axis=1,
                              keepdims=True))
        selmask = selmask + onehot.astype(jnp.float32)
        tmp = jnp.where(onehot, _NEG_INF, tmp)

    topk_idx = jnp.concatenate(idx_cols, axis=1)
    topk_w = jnp.concatenate(w_cols, axis=1)
    denom = jnp.sum(topk_w, axis=1, keepdims=True) + 1e-20
    idx_ref[...] = topk_idx
    tw_ref[...] = topk_w / denom * ROUTED_SCALING

    rowsum = jnp.sum(scores, axis=1, keepdims=True)
    snorm = scores / rowsum
    ssum_ref[...] = jnp.sum(snorm, axis=0, keepdims=True)[None]
    cnt_ref[...] = jnp.sum(selmask, axis=0, keepdims=True)[None]


@functools.partial(jax.jit, static_argnames=())
def kernel(hidden_states, weight, e_score_correction_bias):
    bsz, seq_len, h = hidden_states.shape
    hs = hidden_states.reshape(T, h).astype(jnp.float32)
    w = weight.astype(jnp.float32)
    bias = e_score_correction_bias.astype(jnp.float32).reshape(1, E)

    grid = (NBLK,)
    out_shapes = (
        jax.ShapeDtypeStruct((T, TOP_K), jnp.int32),
        jax.ShapeDtypeStruct((T, TOP_K), jnp.float32),
        jax.ShapeDtypeStruct((NBLK, 1, E), jnp.float32),
        jax.ShapeDtypeStruct((NBLK, 1, E), jnp.float32),
    )
    topk_idx, topk_weight, cnt, ssum = pl.pallas_call(
        _gate_kernel,
        grid=grid,
        in_specs=[
            pl.BlockSpec((BLK, H), lambda i: (i, 0)),
            pl.BlockSpec((E, H), lambda i: (0, 0)),
            pl.BlockSpec((1, E), lambda i: (0, 0)),
        ],
        out_specs=(
            pl.BlockSpec((BLK, TOP_K), lambda i: (i, 0)),
            pl.BlockSpec((BLK, TOP_K), lambda i: (i, 0)),
            pl.BlockSpec((1, 1, E), lambda i: (i, 0, 0)),
            pl.BlockSpec((1, 1, E), lambda i: (i, 0, 0)),
        ),
        out_shape=out_shapes,
    )(hs, w, bias)

    blocks_per_batch = NBLK // bsz
    cnt2 = cnt.reshape(bsz, blocks_per_batch, E).sum(axis=1)
    ssum2 = ssum.reshape(bsz, blocks_per_batch, E).sum(axis=1)
    mean_scores = ssum2 / seq_len
    ce = cnt2 * (E / (seq_len * TOP_K))
    aux_loss = (ce * mean_scores).sum(axis=1).mean() * MOE_LOSS_ALPHA
    num_local_tokens_per_expert = cnt2.sum(axis=0).astype(jnp.int32)
    return topk_idx, topk_weight, aux_loss, num_local_tokens_per_expert


# speedup vs baseline: 1.0160x; 1.0013x over previous
import functools

import jax
import jax.numpy as jnp
from jax.experimental import pallas as pl
from jax.experimental.pallas import tpu as pltpu

BSZ, SEQ, H = 2, 4096, 4096
E = 64
TOP_K = 8
N_GROUP = 8
GSIZE = E // N_GROUP
TOPK_GROUP = 4
ROUTED_SCALING = 2.5
MOE_LOSS_ALPHA = 0.001

BLK = 256
T = BSZ * SEQ
NBLK = T // BLK

_NEG_INF = float("-inf")


def _gate_kernel(x_ref, w_ref, bias_ref, idx_ref, tw_ref, cnt_ref, ssum_ref):
    x = x_ref[...]
    w = w_ref[...]
    logits = jax.lax.dot_general(
        x, w, dimension_numbers=(((1,), (1,)), ((), ())),
        preferred_element_type=jnp.float32)
    scores = jax.nn.sigmoid(logits)
    sc = scores + bias_ref[...]

    iota_g = jax.lax.broadcasted_iota(jnp.int32, (BLK, GSIZE), 1)
    iota_ng = jax.lax.broadcasted_iota(jnp.int32, (BLK, N_GROUP), 1)
    iota_e = jax.lax.broadcasted_iota(jnp.int32, (BLK, E), 1)

    gparts = []
    for g in range(N_GROUP):
        sub = sc[:, g * GSIZE:(g + 1) * GSIZE]
        m1 = jnp.max(sub, axis=1, keepdims=True)
        i1 = jnp.min(jnp.where(sub == m1, iota_g, GSIZE), axis=1, keepdims=True)
        sub2 = jnp.where(iota_g == i1, _NEG_INF, sub)
        m2 = jnp.max(sub2, axis=1, keepdims=True)
        gparts.append(m1 + m2)
    group_scores = jnp.concatenate(gparts, axis=1)

    keep = []
    for g in range(N_GROUP):
        gi = group_scores[:, g:g + 1]
        better = (group_scores > gi) | ((group_scores == gi) & (iota_ng < g))
        rank = jnp.sum(better.astype(jnp.int32), axis=1, keepdims=True)
        keep.append((rank < TOPK_GROUP).astype(jnp.float32))
    emask = jnp.concatenate(
        [jnp.broadcast_to(keep[g], (BLK, GSIZE)) for g in range(N_GROUP)],
        axis=1)

    tmp = jnp.where(emask > 0.5, sc, 0.0)

    idx_cols, w_cols = [], []
    selmask = jnp.zeros((BLK, E), jnp.float32)
    for _ in range(TOP_K):
        m = jnp.max(tmp, axis=1, keepdims=True)
        sel = jnp.min(jnp.where(tmp == m, iota_e, E), axis=1, keepdims=True)
        onehot = iota_e == sel
        idx_cols.append(sel)
        w_cols.append(jnp.sum(jnp.where(onehot, scores, 0.0), axis=1,
                              keepdims=True))
        selmask = selmask + onehot.astype(jnp.float32)
        tmp = jnp.where(onehot, _NEG_INF, tmp)

    topk_idx = jnp.concatenate(idx_cols, axis=1)
    topk_w = jnp.concatenate(w_cols, axis=1)
    denom = jnp.sum(topk_w, axis=1, keepdims=True) + 1e-20
    idx_ref[...] = topk_idx
    tw_ref[...] = topk_w / denom * ROUTED_SCALING

    rowsum = jnp.sum(scores, axis=1, keepdims=True)
    snorm = scores / rowsum
    ssum_ref[...] = jnp.sum(snorm, axis=0, keepdims=True)[None]
    cnt_ref[...] = jnp.sum(selmask, axis=0, keepdims=True)[None]


@functools.partial(jax.jit, static_argnames=())
def kernel(hidden_states, weight, e_score_correction_bias):
    bsz, seq_len, h = hidden_states.shape
    hs = hidden_states.reshape(T, h).astype(jnp.float32)
    w = weight.astype(jnp.float32)
    bias = e_score_correction_bias.astype(jnp.float32).reshape(1, E)

    grid = (NBLK,)
    out_shapes = (
        jax.ShapeDtypeStruct((T, TOP_K), jnp.int32),
        jax.ShapeDtypeStruct((T, TOP_K), jnp.float32),
        jax.ShapeDtypeStruct((NBLK, 1, E), jnp.float32),
        jax.ShapeDtypeStruct((NBLK, 1, E), jnp.float32),
    )
    topk_idx, topk_weight, cnt, ssum = pl.pallas_call(
        _gate_kernel,
        grid=grid,
        in_specs=[
            pl.BlockSpec((BLK, H), lambda i: (i, 0)),
            pl.BlockSpec((E, H), lambda i: (0, 0)),
            pl.BlockSpec((1, E), lambda i: (0, 0)),
        ],
        out_specs=(
            pl.BlockSpec((BLK, TOP_K), lambda i: (i, 0)),
            pl.BlockSpec((BLK, TOP_K), lambda i: (i, 0)),
            pl.BlockSpec((1, 1, E), lambda i: (i, 0, 0)),
            pl.BlockSpec((1, 1, E), lambda i: (i, 0, 0)),
        ),
        out_shape=out_shapes,
        compiler_params=pltpu.CompilerParams(
            dimension_semantics=("parallel",)),
    )(hs, w, bias)

    blocks_per_batch = NBLK // bsz
    cnt2 = cnt.reshape(bsz, blocks_per_batch, E).sum(axis=1)
    ssum2 = ssum.reshape(bsz, blocks_per_batch, E).sum(axis=1)
    mean_scores = ssum2 / seq_len
    ce = cnt2 * (E / (seq_len * TOP_K))
    aux_loss = (ce * mean_scores).sum(axis=1).mean() * MOE_LOSS_ALPHA
    num_local_tokens_per_expert = cnt2.sum(axis=0).astype(jnp.int32)
    return topk_idx, topk_weight, aux_loss, num_local_tokens_per_expert


# speedup vs baseline: 4.3279x; 4.2596x over previous
import functools

import jax
import jax.numpy as jnp
from jax.experimental import pallas as pl
from jax.experimental.pallas import tpu as pltpu

BSZ, SEQ, H = 2, 4096, 4096
E = 64
TOP_K = 8
N_GROUP = 8
GSIZE = E // N_GROUP
TOPK_GROUP = 4
ROUTED_SCALING = 2.5
MOE_LOSS_ALPHA = 0.001

BLK = 256
T = BSZ * SEQ
NBLK = T // BLK

_NEG_INF = float("-inf")


def _gate_kernel(x_ref, w_ref, bias_ref, idx_ref, tw_ref, cnt_ref, ssum_ref):
    x = x_ref[...]
    w = w_ref[...]
    logits = jax.lax.dot_general(
        w, x, dimension_numbers=(((1,), (1,)), ((), ())),
        preferred_element_type=jnp.float32)
    scores = jax.nn.sigmoid(logits)
    sc = scores + bias_ref[...]

    iota_g = jax.lax.broadcasted_iota(jnp.int32, (GSIZE, BLK), 0)
    iota_ng = jax.lax.broadcasted_iota(jnp.int32, (N_GROUP, BLK), 0)
    iota_e = jax.lax.broadcasted_iota(jnp.int32, (E, BLK), 0)

    gparts = []
    for g in range(N_GROUP):
        sub = sc[g * GSIZE:(g + 1) * GSIZE, :]
        m1 = jnp.max(sub, axis=0, keepdims=True)
        i1 = jnp.min(jnp.where(sub == m1, iota_g, GSIZE), axis=0, keepdims=True)
        sub2 = jnp.where(iota_g == i1, _NEG_INF, sub)
        m2 = jnp.max(sub2, axis=0, keepdims=True)
        gparts.append(m1 + m2)
    group_scores = jnp.concatenate(gparts, axis=0)

    keep = []
    for g in range(N_GROUP):
        gi = group_scores[g:g + 1, :]
        better = (group_scores > gi) | ((group_scores == gi) & (iota_ng < g))
        rank = jnp.sum(better.astype(jnp.int32), axis=0, keepdims=True)
        keep.append((rank < TOPK_GROUP).astype(jnp.float32))
    emask = jnp.concatenate(
        [jnp.broadcast_to(keep[g], (GSIZE, BLK)) for g in range(N_GROUP)],
        axis=0)

    tmp = jnp.where(emask > 0.5, sc, 0.0)

    idx_rows, w_rows = [], []
    selmask = jnp.zeros((E, BLK), jnp.float32)
    for _ in range(TOP_K):
        m = jnp.max(tmp, axis=0, keepdims=True)
        sel = jnp.min(jnp.where(tmp == m, iota_e, E), axis=0, keepdims=True)
        onehot = iota_e == sel
        idx_rows.append(sel)
        w_rows.append(jnp.sum(jnp.where(onehot, scores, 0.0), axis=0,
                              keepdims=True))
        selmask = selmask + onehot.astype(jnp.float32)
        tmp = jnp.where(onehot, _NEG_INF, tmp)

    topk_idx = jnp.concatenate(idx_rows, axis=0)
    topk_w = jnp.concatenate(w_rows, axis=0)
    denom = jnp.sum(topk_w, axis=0, keepdims=True) + 1e-20
    idx_ref[...] = topk_idx
    tw_ref[...] = topk_w / denom * ROUTED_SCALING

    rowsum = jnp.sum(scores, axis=0, keepdims=True)
    snorm = scores / rowsum
    ssum_ref[...] = jnp.sum(snorm, axis=1, keepdims=True)[None]
    cnt_ref[...] = jnp.sum(selmask, axis=1, keepdims=True)[None]


@functools.partial(jax.jit, static_argnames=())
def kernel(hidden_states, weight, e_score_correction_bias):
    bsz, seq_len, h = hidden_states.shape
    hs = hidden_states.reshape(T, h).astype(jnp.float32)
    w = weight.astype(jnp.float32)
    bias = e_score_correction_bias.astype(jnp.float32).reshape(E, 1)

    grid = (NBLK,)
    out_shapes = (
        jax.ShapeDtypeStruct((TOP_K, T), jnp.int32),
        jax.ShapeDtypeStruct((TOP_K, T), jnp.float32),
        jax.ShapeDtypeStruct((NBLK, E, 1), jnp.float32),
        jax.ShapeDtypeStruct((NBLK, E, 1), jnp.float32),
    )
    idx_t, tw_t, cnt, ssum = pl.pallas_call(
        _gate_kernel,
        grid=grid,
        in_specs=[
            pl.BlockSpec((BLK, H), lambda i: (i, 0)),
            pl.BlockSpec((E, H), lambda i: (0, 0)),
            pl.BlockSpec((E, 1), lambda i: (0, 0)),
        ],
        out_specs=(
            pl.BlockSpec((TOP_K, BLK), lambda i: (0, i)),
            pl.BlockSpec((TOP_K, BLK), lambda i: (0, i)),
            pl.BlockSpec((1, E, 1), lambda i: (i, 0, 0)),
            pl.BlockSpec((1, E, 1), lambda i: (i, 0, 0)),
        ),
        out_shape=out_shapes,
        compiler_params=pltpu.CompilerParams(
            dimension_semantics=("parallel",)),
    )(hs, w, bias)

    topk_idx = idx_t.T
    topk_weight = tw_t.T

    blocks_per_batch = NBLK // bsz
    cnt2 = cnt.reshape(bsz, blocks_per_batch, E).sum(axis=1)
    ssum2 = ssum.reshape(bsz, blocks_per_batch, E).sum(axis=1)
    mean_scores = ssum2 / seq_len
    ce = cnt2 * (E / (seq_len * TOP_K))
    aux_loss = (ce * mean_scores).sum(axis=1).mean() * MOE_LOSS_ALPHA
    num_local_tokens_per_expert = cnt2.sum(axis=0).astype(jnp.int32)
    return topk_idx, topk_weight, aux_loss, num_local_tokens_per_expert


# speedup vs baseline: 5.1232x; 1.1838x over previous
import functools

import jax
import jax.numpy as jnp
from jax.experimental import pallas as pl
from jax.experimental.pallas import tpu as pltpu

BSZ, SEQ, H = 2, 4096, 4096
E = 64
TOP_K = 8
N_GROUP = 8
GSIZE = E // N_GROUP
TOPK_GROUP = 4
ROUTED_SCALING = 2.5
MOE_LOSS_ALPHA = 0.001

BLK = 512
T = BSZ * SEQ
NBLK = T // BLK

_NEG_INF = float("-inf")


def _gate_kernel(x_ref, w_ref, bias_ref, idx_ref, tw_ref, cnt_ref, ssum_ref):
    x = x_ref[...]
    w = w_ref[...]
    logits = jax.lax.dot_general(
        w, x, dimension_numbers=(((1,), (1,)), ((), ())),
        preferred_element_type=jnp.float32)
    scores = jax.nn.sigmoid(logits)
    sc = scores + bias_ref[...]

    iota_g = jax.lax.broadcasted_iota(jnp.int32, (GSIZE, BLK), 0)
    iota_ng = jax.lax.broadcasted_iota(jnp.int32, (N_GROUP, BLK), 0)
    iota_e = jax.lax.broadcasted_iota(jnp.int32, (E, BLK), 0)

    gparts = []
    for g in range(N_GROUP):
        sub = sc[g * GSIZE:(g + 1) * GSIZE, :]
        m1 = jnp.max(sub, axis=0, keepdims=True)
        i1 = jnp.min(jnp.where(sub == m1, iota_g, GSIZE), axis=0, keepdims=True)
        sub2 = jnp.where(iota_g == i1, _NEG_INF, sub)
        m2 = jnp.max(sub2, axis=0, keepdims=True)
        gparts.append(m1 + m2)
    group_scores = jnp.concatenate(gparts, axis=0)

    keep = []
    for g in range(N_GROUP):
        gi = group_scores[g:g + 1, :]
        better = (group_scores > gi) | ((group_scores == gi) & (iota_ng < g))
        rank = jnp.sum(better.astype(jnp.int32), axis=0, keepdims=True)
        keep.append((rank < TOPK_GROUP).astype(jnp.float32))
    emask = jnp.concatenate(
        [jnp.broadcast_to(keep[g], (GSIZE, BLK)) for g in range(N_GROUP)],
        axis=0)

    tmp = jnp.where(emask > 0.5, sc, 0.0)

    idx_rows, w_rows = [], []
    selmask = jnp.zeros((E, BLK), jnp.float32)
    for _ in range(TOP_K):
        m = jnp.max(tmp, axis=0, keepdims=True)
        sel = jnp.min(jnp.where(tmp == m, iota_e, E), axis=0, keepdims=True)
        onehot = iota_e == sel
        idx_rows.append(sel)
        w_rows.append(jnp.sum(jnp.where(onehot, scores, 0.0), axis=0,
                              keepdims=True))
        selmask = selmask + onehot.astype(jnp.float32)
        tmp = jnp.where(onehot, _NEG_INF, tmp)

    topk_idx = jnp.concatenate(idx_rows, axis=0)
    topk_w = jnp.concatenate(w_rows, axis=0)
    denom = jnp.sum(topk_w, axis=0, keepdims=True) + 1e-20
    idx_ref[...] = topk_idx
    tw_ref[...] = topk_w / denom * ROUTED_SCALING

    rowsum = jnp.sum(scores, axis=0, keepdims=True)
    snorm = scores / rowsum
    ssum_ref[...] = jnp.sum(snorm, axis=1, keepdims=True)[None]
    cnt_ref[...] = jnp.sum(selmask, axis=1, keepdims=True)[None]


@functools.partial(jax.jit, static_argnames=())
def kernel(hidden_states, weight, e_score_correction_bias):
    bsz, seq_len, h = hidden_states.shape
    hs = hidden_states.reshape(T, h).astype(jnp.float32)
    w = weight.astype(jnp.float32)
    bias = e_score_correction_bias.astype(jnp.float32).reshape(E, 1)

    grid = (NBLK,)
    out_shapes = (
        jax.ShapeDtypeStruct((TOP_K, T), jnp.int32),
        jax.ShapeDtypeStruct((TOP_K, T), jnp.float32),
        jax.ShapeDtypeStruct((NBLK, E, 1), jnp.float32),
        jax.ShapeDtypeStruct((NBLK, E, 1), jnp.float32),
    )
    idx_t, tw_t, cnt, ssum = pl.pallas_call(
        _gate_kernel,
        grid=grid,
        in_specs=[
            pl.BlockSpec((BLK, H), lambda i: (i, 0)),
            pl.BlockSpec((E, H), lambda i: (0, 0)),
            pl.BlockSpec((E, 1), lambda i: (0, 0)),
        ],
        out_specs=(
            pl.BlockSpec((TOP_K, BLK), lambda i: (0, i)),
            pl.BlockSpec((TOP_K, BLK), lambda i: (0, i)),
            pl.BlockSpec((1, E, 1), lambda i: (i, 0, 0)),
            pl.BlockSpec((1, E, 1), lambda i: (i, 0, 0)),
        ),
        out_shape=out_shapes,
        compiler_params=pltpu.CompilerParams(
            dimension_semantics=("parallel",)),
    )(hs, w, bias)

    topk_idx = idx_t.T
    topk_weight = tw_t.T

    blocks_per_batch = NBLK // bsz
    cnt2 = cnt.reshape(bsz, blocks_per_batch, E).sum(axis=1)
    ssum2 = ssum.reshape(bsz, blocks_per_batch, E).sum(axis=1)
    mean_scores = ssum2 / seq_len
    ce = cnt2 * (E / (seq_len * TOP_K))
    aux_loss = (ce * mean_scores).sum(axis=1).mean() * MOE_LOSS_ALPHA
    num_local_tokens_per_expert = cnt2.sum(axis=0).astype(jnp.int32)
    return topk_idx, topk_weight, aux_loss, num_local_tokens_per_expert


# speedup vs baseline: 5.5760x; 1.0884x over previous
import functools

import jax
import jax.numpy as jnp
from jax.experimental import pallas as pl
from jax.experimental.pallas import tpu as pltpu

BSZ, SEQ, H = 2, 4096, 4096
E = 64
TOP_K = 8
N_GROUP = 8
GSIZE = E // N_GROUP
TOPK_GROUP = 4
ROUTED_SCALING = 2.5
MOE_LOSS_ALPHA = 0.001

BLK = 1024
T = BSZ * SEQ
NBLK = T // BLK

_NEG_INF = float("-inf")


def _gate_kernel(x_ref, w_ref, bias_ref, idx_ref, tw_ref, cnt_ref, ssum_ref):
    x = x_ref[...]
    w = w_ref[...]
    logits = jax.lax.dot_general(
        w, x, dimension_numbers=(((1,), (1,)), ((), ())),
        preferred_element_type=jnp.float32)
    scores = jax.nn.sigmoid(logits)
    sc = scores + bias_ref[...]

    iota_g = jax.lax.broadcasted_iota(jnp.int32, (GSIZE, BLK), 0)
    iota_ng = jax.lax.broadcasted_iota(jnp.int32, (N_GROUP, BLK), 0)
    iota_e = jax.lax.broadcasted_iota(jnp.int32, (E, BLK), 0)

    gparts = []
    for g in range(N_GROUP):
        sub = sc[g * GSIZE:(g + 1) * GSIZE, :]
        m1 = jnp.max(sub, axis=0, keepdims=True)
        i1 = jnp.min(jnp.where(sub == m1, iota_g, GSIZE), axis=0, keepdims=True)
        sub2 = jnp.where(iota_g == i1, _NEG_INF, sub)
        m2 = jnp.max(sub2, axis=0, keepdims=True)
        gparts.append(m1 + m2)
    group_scores = jnp.concatenate(gparts, axis=0)

    keep = []
    for g in range(N_GROUP):
        gi = group_scores[g:g + 1, :]
        better = (group_scores > gi) | ((group_scores == gi) & (iota_ng < g))
        rank = jnp.sum(better.astype(jnp.int32), axis=0, keepdims=True)
        keep.append((rank < TOPK_GROUP).astype(jnp.float32))
    emask = jnp.concatenate(
        [jnp.broadcast_to(keep[g], (GSIZE, BLK)) for g in range(N_GROUP)],
        axis=0)

    tmp = jnp.where(emask > 0.5, sc, 0.0)

    idx_rows, w_rows = [], []
    selmask = jnp.zeros((E, BLK), jnp.float32)
    for _ in range(TOP_K):
        m = jnp.max(tmp, axis=0, keepdims=True)
        sel = jnp.min(jnp.where(tmp == m, iota_e, E), axis=0, keepdims=True)
        onehot = iota_e == sel
        idx_rows.append(sel)
        w_rows.append(jnp.sum(jnp.where(onehot, scores, 0.0), axis=0,
                              keepdims=True))
        selmask = selmask + onehot.astype(jnp.float32)
        tmp = jnp.where(onehot, _NEG_INF, tmp)

    topk_idx = jnp.concatenate(idx_rows, axis=0)
    topk_w = jnp.concatenate(w_rows, axis=0)
    denom = jnp.sum(topk_w, axis=0, keepdims=True) + 1e-20
    idx_ref[...] = topk_idx
    tw_ref[...] = topk_w / denom * ROUTED_SCALING

    rowsum = jnp.sum(scores, axis=0, keepdims=True)
    snorm = scores / rowsum
    ssum_ref[...] = jnp.sum(snorm, axis=1, keepdims=True)[None]
    cnt_ref[...] = jnp.sum(selmask, axis=1, keepdims=True)[None]


@functools.partial(jax.jit, static_argnames=())
def kernel(hidden_states, weight, e_score_correction_bias):
    bsz, seq_len, h = hidden_states.shape
    hs = hidden_states.reshape(T, h).astype(jnp.float32)
    w = weight.astype(jnp.float32)
    bias = e_score_correction_bias.astype(jnp.float32).reshape(E, 1)

    grid = (NBLK,)
    out_shapes = (
        jax.ShapeDtypeStruct((TOP_K, T), jnp.int32),
        jax.ShapeDtypeStruct((TOP_K, T), jnp.float32),
        jax.ShapeDtypeStruct((NBLK, E, 1), jnp.float32),
        jax.ShapeDtypeStruct((NBLK, E, 1), jnp.float32),
    )
    idx_t, tw_t, cnt, ssum = pl.pallas_call(
        _gate_kernel,
        grid=grid,
        in_specs=[
            pl.BlockSpec((BLK, H), lambda i: (i, 0)),
            pl.BlockSpec((E, H), lambda i: (0, 0)),
            pl.BlockSpec((E, 1), lambda i: (0, 0)),
        ],
        out_specs=(
            pl.BlockSpec((TOP_K, BLK), lambda i: (0, i)),
            pl.BlockSpec((TOP_K, BLK), lambda i: (0, i)),
            pl.BlockSpec((1, E, 1), lambda i: (i, 0, 0)),
            pl.BlockSpec((1, E, 1), lambda i: (i, 0, 0)),
        ),
        out_shape=out_shapes,
        compiler_params=pltpu.CompilerParams(
            dimension_semantics=("parallel",)),
    )(hs, w, bias)

    topk_idx = idx_t.T
    topk_weight = tw_t.T

    blocks_per_batch = NBLK // bsz
    cnt2 = cnt.reshape(bsz, blocks_per_batch, E).sum(axis=1)
    ssum2 = ssum.reshape(bsz, blocks_per_batch, E).sum(axis=1)
    mean_scores = ssum2 / seq_len
    ce = cnt2 * (E / (seq_len * TOP_K))
    aux_loss = (ce * mean_scores).sum(axis=1).mean() * MOE_LOSS_ALPHA
    num_local_tokens_per_expert = cnt2.sum(axis=0).astype(jnp.int32)
    return topk_idx, topk_weight, aux_loss, num_local_tokens_per_expert
